# Initial kernel scaffold; baseline (speedup 1.0000x reference)
import jax
import jax.numpy as jnp
from jax.experimental import pallas as pl


def kernel(x, edge_index, W, gamma, beta):
    raise NotImplementedError("write your pallas kernel here")


# speedup vs baseline: 5.4794x; 5.4794x over previous
import functools

import jax
import jax.numpy as jnp
from jax import lax
from jax.experimental import pallas as pl
from jax.experimental.pallas import tpu as pltpu
from jax.experimental.pallas import tpu_sc as plsc

_N = 10000
_E = 320000
_D = 128
_EPS = 1e-5

_NC = 2
_NS = 16
_NW = _NC * _NS
_CH = 80
_EPW = _E // _NW
_NCHUNK = _EPW // _CH
_RPT = 624
_RTAIL = _N - _NS * _RPT


def _matmul_body(x_ref, w_ref, z_ref):
    z_ref[...] = lax.dot_general(
        x_ref[...], w_ref[...],
        dimension_numbers=(((1,), (1,)), ((), ())),
        preferred_element_type=jnp.float32,
    )


def _project(x, W):
    blk = 2000
    return pl.pallas_call(
        _matmul_body,
        grid=(_N // blk,),
        in_specs=[
            pl.BlockSpec((blk, _D), lambda i: (i, 0)),
            pl.BlockSpec((_D, _D), lambda i: (0, 0)),
        ],
        out_specs=pl.BlockSpec((blk, _D), lambda i: (i, 0)),
        out_shape=jax.ShapeDtypeStruct((_N, _D), jnp.float32),
    )(x, W)


def _sc_body(z_hbm, src_hbm, dst_hbm, out_hbm, srcv, dstv, rows, hpart):
    cid = lax.axis_index("core")
    sid = lax.axis_index("subcore")

    @pl.loop(0, _CH)
    def _zero_rows(i):
        @pl.loop(0, _D, step=16)
        def _zero_lane(j):
            rows[i, pl.ds(j, 16)] = jnp.zeros((16,), jnp.float32)

    row0 = sid * _RPT
    @pl.loop(0, _RPT // _CH)
    def _zero_hpart(k):
        pltpu.sync_copy(rows, hpart.at[pl.ds(row0 + k * _CH, _CH)])
    _tail = _RPT % _CH
    pltpu.sync_copy(rows.at[pl.ds(0, _tail)],
                    hpart.at[pl.ds(row0 + _RPT - _tail, _tail)])

    @pl.when(sid == _NS - 1)
    def _zero_last():
        pltpu.sync_copy(rows.at[pl.ds(0, _RTAIL)],
                        hpart.at[pl.ds(_NS * _RPT, _RTAIL)])

    plsc.subcore_barrier()

    base0 = (cid * _NS + sid) * _EPW

    @pl.loop(0, _NCHUNK)
    def _edge_chunk(j):
        b = base0 + j * _CH
        pltpu.sync_copy(src_hbm.at[pl.ds(b, _CH)], srcv)
        pltpu.sync_copy(dst_hbm.at[pl.ds(b, _CH)], dstv)
        pltpu.sync_copy(z_hbm.at[srcv], rows)
        pltpu.sync_copy(rows, hpart.at[dstv], add=True)

    plsc.subcore_barrier()

    pltpu.sync_copy(hpart.at[pl.ds(row0, _RPT)],
                    out_hbm.at[cid, pl.ds(row0, _RPT)])

    @pl.when(sid == _NS - 1)
    def _write_last():
        pltpu.sync_copy(hpart.at[pl.ds(_NS * _RPT, _RTAIL)],
                        out_hbm.at[cid, pl.ds(_NS * _RPT, _RTAIL)])


def _sc_aggregate(z, src, dst):
    mesh = plsc.VectorSubcoreMesh(core_axis_name="core",
                                  subcore_axis_name="subcore")
    f = pl.kernel(
        _sc_body,
        out_type=jax.ShapeDtypeStruct((_NC, _N, _D), jnp.float32),
        mesh=mesh,
        scratch_types=[
            pltpu.VMEM((_CH,), jnp.int32),
            pltpu.VMEM((_CH,), jnp.int32),
            pltpu.VMEM((_CH, _D), jnp.float32),
            pltpu.VMEM_SHARED((_N, _D), jnp.float32),
        ],
    )
    return f(z, src, dst)


def _bn_body(p_ref, g_ref, b_ref, o_ref):
    h = p_ref[0] + p_ref[1]
    mean = jnp.mean(h, axis=0, keepdims=True)
    c = h - mean
    var = jnp.mean(c * c, axis=0, keepdims=True)
    hn = c * lax.rsqrt(var + _EPS) * g_ref[...] + b_ref[...]
    o_ref[...] = jnp.where(hn > 0, hn, jnp.exp(jnp.minimum(hn, 0.0)) - 1.0)


def _bn_elu(parts, gamma, beta):
    return pl.pallas_call(
        _bn_body,
        out_shape=jax.ShapeDtypeStruct((_N, _D), jnp.float32),
    )(parts, gamma.reshape(1, _D), beta.reshape(1, _D))


def kernel(x, edge_index, W, gamma, beta):
    z = _project(x, W)
    parts = _sc_aggregate(z, edge_index[0], edge_index[1])
    return _bn_elu(parts, gamma, beta)


# speedup vs baseline: 9.5665x; 1.7459x over previous
import jax
import jax.numpy as jnp
from jax import lax
from jax.experimental import pallas as pl
from jax.experimental.pallas import tpu as pltpu
from jax.experimental.pallas import tpu_sc as plsc

_N = 10000
_E = 320000
_D = 128
_EPS = 1e-5

_NC = 2
_NS = 16
_NW = _NC * _NS
_CH = 80
_EPW = _E // _NW
_NCHUNK = _EPW // _CH
_RPT = 624
_RTAIL = _N - _NS * _RPT


def _matmul_body(x_ref, w_ref, z_ref):
    z_ref[...] = lax.dot_general(
        x_ref[...], w_ref[...],
        dimension_numbers=(((1,), (1,)), ((), ())),
        preferred_element_type=jnp.float32,
    )


def _project(x, W):
    blk = 2000
    return pl.pallas_call(
        _matmul_body,
        grid=(_N // blk,),
        in_specs=[
            pl.BlockSpec((blk, _D), lambda i: (i, 0)),
            pl.BlockSpec((_D, _D), lambda i: (0, 0)),
        ],
        out_specs=pl.BlockSpec((blk, _D), lambda i: (i, 0)),
        out_shape=jax.ShapeDtypeStruct((_N, _D), jnp.float32),
    )(x, W)


def _sc_body(z_hbm, src_hbm, dst_hbm, out_hbm,
             srcs, dsts, rows0, rows1, hpart,
             gsem0, gsem1, ssem0, ssem1):
    cid = lax.axis_index("core")
    sid = lax.axis_index("subcore")
    wid = cid * _NS + sid

    pltpu.sync_copy(src_hbm.at[wid], srcs)
    pltpu.sync_copy(dst_hbm.at[wid], dsts)

    @pl.loop(0, _CH)
    def _zero_rows(i):
        @pl.loop(0, _D, step=16)
        def _zero_lane(j):
            rows0[i, pl.ds(j, 16)] = jnp.zeros((16,), jnp.float32)

    row0 = sid * _RPT
    @pl.loop(0, _RPT // _CH)
    def _zero_hpart(k):
        pltpu.sync_copy(rows0, hpart.at[pl.ds(row0 + k * _CH, _CH)])
    _tail = _RPT % _CH
    pltpu.sync_copy(rows0.at[pl.ds(0, _tail)],
                    hpart.at[pl.ds(row0 + _RPT - _tail, _tail)])

    @pl.when(sid == _NS - 1)
    def _zero_last():
        pltpu.sync_copy(rows0.at[pl.ds(0, _RTAIL)],
                        hpart.at[pl.ds(_NS * _RPT, _RTAIL)])

    plsc.subcore_barrier()

    def _src_slice(j):
        return srcs.at[pl.ds(j * _CH, _CH)]

    pltpu.async_copy(z_hbm.at[_src_slice(0)], rows0, gsem0)
    pltpu.async_copy(z_hbm.at[_src_slice(1)], rows1, gsem1)

    @pl.loop(0, _NCHUNK // 2)
    def _pair(k):
        j = 2 * k
        pltpu.make_async_copy(z_hbm.at[_src_slice(j)], rows0, gsem0).wait()
        pltpu.async_copy(rows0, hpart.at[dsts.at[j]], ssem0, add=True)
        pltpu.make_async_copy(z_hbm.at[_src_slice(j + 1)], rows1, gsem1).wait()
        pltpu.async_copy(rows1, hpart.at[dsts.at[j + 1]], ssem1, add=True)
        pltpu.make_async_copy(rows0, hpart.at[dsts.at[j]], ssem0).wait()
        @pl.when(j + 2 < _NCHUNK)
        def _next0():
            pltpu.async_copy(z_hbm.at[_src_slice(j + 2)], rows0, gsem0)
        pltpu.make_async_copy(rows1, hpart.at[dsts.at[j + 1]], ssem1).wait()
        @pl.when(j + 3 < _NCHUNK)
        def _next1():
            pltpu.async_copy(z_hbm.at[_src_slice(j + 3)], rows1, gsem1)

    _last = _NCHUNK - 1
    pltpu.make_async_copy(z_hbm.at[_src_slice(_last)], rows0, gsem0).wait()
    pltpu.sync_copy(rows0, hpart.at[dsts.at[_last]], add=True)

    plsc.subcore_barrier()

    pltpu.sync_copy(hpart.at[pl.ds(row0, _RPT)],
                    out_hbm.at[cid, pl.ds(row0, _RPT)])

    @pl.when(sid == _NS - 1)
    def _write_last():
        pltpu.sync_copy(hpart.at[pl.ds(_NS * _RPT, _RTAIL)],
                        out_hbm.at[cid, pl.ds(_NS * _RPT, _RTAIL)])


def _sc_aggregate(z, src, dst):
    mesh = plsc.VectorSubcoreMesh(core_axis_name="core",
                                  subcore_axis_name="subcore")
    f = pl.kernel(
        _sc_body,
        out_type=jax.ShapeDtypeStruct((_NC, _N, _D), jnp.float32),
        mesh=mesh,
        scratch_types=[
            pltpu.VMEM((_EPW,), jnp.int32),
            pltpu.VMEM((_NCHUNK, _CH), jnp.int32),
            pltpu.VMEM((_CH, _D), jnp.float32),
            pltpu.VMEM((_CH, _D), jnp.float32),
            pltpu.VMEM_SHARED((_N, _D), jnp.float32),
            pltpu.SemaphoreType.DMA,
            pltpu.SemaphoreType.DMA,
            pltpu.SemaphoreType.DMA,
            pltpu.SemaphoreType.DMA,
        ],
    )
    return f(z, src.reshape(_NW, _EPW), dst.reshape(_NW, _NCHUNK, _CH))


def _bn_body(p_ref, g_ref, b_ref, o_ref):
    h = p_ref[0] + p_ref[1]
    mean = jnp.mean(h, axis=0, keepdims=True)
    c = h - mean
    var = jnp.mean(c * c, axis=0, keepdims=True)
    hn = c * lax.rsqrt(var + _EPS) * g_ref[...] + b_ref[...]
    o_ref[...] = jnp.where(hn > 0, hn, jnp.exp(jnp.minimum(hn, 0.0)) - 1.0)


def _bn_elu(parts, gamma, beta):
    return pl.pallas_call(
        _bn_body,
        out_shape=jax.ShapeDtypeStruct((_N, _D), jnp.float32),
    )(parts, gamma.reshape(1, _D), beta.reshape(1, _D))


def kernel(x, edge_index, W, gamma, beta):
    z = _project(x, W)
    parts = _sc_aggregate(z, edge_index[0], edge_index[1])
    return _bn_elu(parts, gamma, beta)


# speedup vs baseline: 9.5825x; 1.0017x over previous
import jax
import jax.numpy as jnp
from jax import lax
from jax.experimental import pallas as pl
from jax.experimental.pallas import tpu as pltpu
from jax.experimental.pallas import tpu_sc as plsc

_N = 10000
_E = 320000
_D = 128
_EPS = 1e-5

_NC = 2
_NS = 16
_NW = _NC * _NS
_CH = 80
_EPW = _E // _NW
_NCHUNK = _EPW // _CH
_RPT = 624
_RTAIL = _N - _NS * _RPT


def _matmul_body(x_ref, w_ref, z_ref):
    z_ref[...] = lax.dot_general(
        x_ref[...].astype(jnp.bfloat16), w_ref[...].astype(jnp.bfloat16),
        dimension_numbers=(((1,), (1,)), ((), ())),
        preferred_element_type=jnp.float32,
    )


def _project(x, W):
    blk = 2000
    return pl.pallas_call(
        _matmul_body,
        grid=(_N // blk,),
        in_specs=[
            pl.BlockSpec((blk, _D), lambda i: (i, 0)),
            pl.BlockSpec((_D, _D), lambda i: (0, 0)),
        ],
        out_specs=pl.BlockSpec((blk, _D), lambda i: (i, 0)),
        out_shape=jax.ShapeDtypeStruct((_N, _D), jnp.float32),
    )(x, W)


def _sc_body(z_hbm, src_hbm, dst_hbm, out_hbm,
             srcs, dsts, rows0, rows1, hpart,
             gsem0, gsem1, ssem0, ssem1):
    cid = lax.axis_index("core")
    sid = lax.axis_index("subcore")
    wid = cid * _NS + sid

    pltpu.sync_copy(src_hbm.at[wid], srcs)
    pltpu.sync_copy(dst_hbm.at[wid], dsts)

    @pl.loop(0, _CH)
    def _zero_rows(i):
        @pl.loop(0, _D, step=16)
        def _zero_lane(j):
            rows0[i, pl.ds(j, 16)] = jnp.zeros((16,), jnp.float32)

    row0 = sid * _RPT
    @pl.loop(0, _RPT // _CH)
    def _zero_hpart(k):
        pltpu.sync_copy(rows0, hpart.at[pl.ds(row0 + k * _CH, _CH)])
    _tail = _RPT % _CH
    pltpu.sync_copy(rows0.at[pl.ds(0, _tail)],
                    hpart.at[pl.ds(row0 + _RPT - _tail, _tail)])

    @pl.when(sid == _NS - 1)
    def _zero_last():
        pltpu.sync_copy(rows0.at[pl.ds(0, _RTAIL)],
                        hpart.at[pl.ds(_NS * _RPT, _RTAIL)])

    def _src_slice(j):
        return srcs.at[pl.ds(j * _CH, _CH)]

    pltpu.async_copy(z_hbm.at[_src_slice(0)], rows0, gsem0)
    pltpu.async_copy(z_hbm.at[_src_slice(1)], rows1, gsem1)

    plsc.subcore_barrier()


    @pl.loop(0, _NCHUNK // 2)
    def _pair(k):
        j = 2 * k
        pltpu.make_async_copy(z_hbm.at[_src_slice(j)], rows0, gsem0).wait()
        pltpu.async_copy(rows0, hpart.at[dsts.at[j]], ssem0, add=True)
        pltpu.make_async_copy(z_hbm.at[_src_slice(j + 1)], rows1, gsem1).wait()
        pltpu.async_copy(rows1, hpart.at[dsts.at[j + 1]], ssem1, add=True)
        pltpu.make_async_copy(rows0, hpart.at[dsts.at[j]], ssem0).wait()
        @pl.when(j + 2 < _NCHUNK)
        def _next0():
            pltpu.async_copy(z_hbm.at[_src_slice(j + 2)], rows0, gsem0)
        pltpu.make_async_copy(rows1, hpart.at[dsts.at[j + 1]], ssem1).wait()
        @pl.when(j + 3 < _NCHUNK)
        def _next1():
            pltpu.async_copy(z_hbm.at[_src_slice(j + 3)], rows1, gsem1)

    _last = _NCHUNK - 1
    pltpu.make_async_copy(z_hbm.at[_src_slice(_last)], rows0, gsem0).wait()
    pltpu.sync_copy(rows0, hpart.at[dsts.at[_last]], add=True)

    plsc.subcore_barrier()

    pltpu.sync_copy(hpart.at[pl.ds(row0, _RPT)],
                    out_hbm.at[cid, pl.ds(row0, _RPT)])

    @pl.when(sid == _NS - 1)
    def _write_last():
        pltpu.sync_copy(hpart.at[pl.ds(_NS * _RPT, _RTAIL)],
                        out_hbm.at[cid, pl.ds(_NS * _RPT, _RTAIL)])


def _sc_aggregate(z, src, dst):
    mesh = plsc.VectorSubcoreMesh(core_axis_name="core",
                                  subcore_axis_name="subcore")
    f = pl.kernel(
        _sc_body,
        out_type=jax.ShapeDtypeStruct((_NC, _N, _D), jnp.float32),
        mesh=mesh,
        scratch_types=[
            pltpu.VMEM((_EPW,), jnp.int32),
            pltpu.VMEM((_NCHUNK, _CH), jnp.int32),
            pltpu.VMEM((_CH, _D), jnp.float32),
            pltpu.VMEM((_CH, _D), jnp.float32),
            pltpu.VMEM_SHARED((_N, _D), jnp.float32),
            pltpu.SemaphoreType.DMA,
            pltpu.SemaphoreType.DMA,
            pltpu.SemaphoreType.DMA,
            pltpu.SemaphoreType.DMA,
        ],
    )
    return f(z, src.reshape(_NW, _EPW), dst.reshape(_NW, _NCHUNK, _CH))


def _bn_body(p_ref, g_ref, b_ref, o_ref):
    h = p_ref[0] + p_ref[1]
    mean = jnp.mean(h, axis=0, keepdims=True)
    c = h - mean
    var = jnp.mean(c * c, axis=0, keepdims=True)
    hn = c * lax.rsqrt(var + _EPS) * g_ref[...][None, :] + b_ref[...][None, :]
    o_ref[...] = jnp.where(hn > 0, hn, jnp.exp(jnp.minimum(hn, 0.0)) - 1.0)


def _bn_elu(parts, gamma, beta):
    return pl.pallas_call(
        _bn_body,
        out_shape=jax.ShapeDtypeStruct((_N, _D), jnp.float32),
    )(parts, gamma, beta)


def kernel(x, edge_index, W, gamma, beta):
    z = _project(x, W)
    parts = _sc_aggregate(z, edge_index[0], edge_index[1])
    return _bn_elu(parts, gamma, beta)
